# baseline (speedup 1.0000x reference)
import functools

import jax
import jax.numpy as jnp
from jax import lax
from jax.experimental import pallas as pl
from jax.experimental.pallas import tpu as pltpu
from jax.experimental.pallas import tpu_sc as plsc

VOCAB = 1000000
HIDDEN = 64
SEQ_LEN = 200
BATCH = 4096
N = BATCH * SEQ_LEN
NC, NS, LANES = 2, 16, 16
NW = NC * NS
BBLK = BATCH // NW
HQ = HIDDEN // LANES
CL = 2
CTOK = CL * BBLK
NCHUNK = SEQ_LEN // CL
NGRP = BBLK // LANES
LN_EPS = 1e-8


def _rsqrt(x):
    i = lax.bitcast_convert_type(x, jnp.int32)
    i = jnp.int32(0x5F3759DF) - lax.shift_right_logical(i, 1)
    y = lax.bitcast_convert_type(i, jnp.float32)
    for _ in range(3):
        y = y * (1.5 - 0.5 * x * y * y)
    return y


def _emb_body(tok_hbm, word_hbm, pos_hbm, gam_hbm, bet_hbm, out_hbm,
              idxa, idxb, rowsa, rowsb, outa, outb,
              pos_v, stg, gv, bv, sga, sgb, swa, swb):
    wid = lax.axis_index("s") * NC + lax.axis_index("c")
    b0 = wid * BBLK

    pltpu.sync_copy(pos_hbm, pos_v)
    pltpu.sync_copy(gam_hbm, gv)
    pltpu.sync_copy(bet_hbm, bv)

    gvec = [gv[pl.ds(i * LANES, LANES)] for i in range(HQ)]
    bvec = [bv[pl.ds(i * LANES, LANES)] for i in range(HQ)]
    lane = lax.iota(jnp.int32, LANES)
    lane_hi = lax.shift_right_logical(lane, 3)
    lane_lo = lane & 7
    rowq = [lane_hi + 2 * q for q in range(HQ)]
    zero = lane & 0
    hsp = [zero + h for h in range(LANES)]

    def issue_gather(c, idxf, rows, sem):
        l0 = c * CL
        for li in range(CL):
            pltpu.sync_copy(tok_hbm.at[l0 + li, pl.ds(b0, BBLK)],
                            idxf.at[pl.ds(li * BBLK, BBLK)])
        for li in range(CL):
            pltpu.async_copy(word_hbm.at[idxf.at[pl.ds(li * BBLK, BBLK)]],
                             rows.at[pl.ds(li * BBLK, BBLK)], sem)

    def wait_gather(idxf, rows, sem):
        for li in range(CL):
            pltpu.make_async_copy(word_hbm.at[idxf.at[pl.ds(li * BBLK, BBLK)]],
                                  rows.at[pl.ds(li * BBLK, BBLK)], sem).wait()

    def compute(c, idxf, rows, outv):
        l0 = c * CL
        for li in range(CL):
            pq = [pos_v[pl.ds((l0 + li) * HIDDEN + i * LANES, LANES)]
                  for i in range(HQ)]
            livec = zero + li

            @plsc.parallel_loop(0, NGRP, 1, unroll=1)
            def group_body(g):
                gbase = li * BBLK + g * LANES
                tokv = idxf[pl.ds(gbase, LANES)]
                msk16 = jnp.where(tokv != 0, jnp.float32(1.0),
                                  jnp.float32(0.0))
                for k in range(LANES):
                    x = [rows[gbase + k, pl.ds(i * LANES, LANES)] + pq[i]
                         for i in range(HQ)]
                    sq = (x[0] + x[1]) + (x[2] + x[3])
                    sq2 = (x[0] * x[0] + x[1] * x[1]) \
                        + (x[2] * x[2] + x[3] * x[3])
                    stg[g, 0, k] = sq
                    stg[g, 1, k] = sq2
                gsp = zero + g
                s16 = plsc.load_gather(stg, [gsp, zero, lane, hsp[0]])
                t16 = plsc.load_gather(stg, [gsp, zero + 1, lane, hsp[0]])
                for h in range(1, LANES):
                    s16 = s16 + plsc.load_gather(stg, [gsp, zero, lane, hsp[h]])
                    t16 = t16 + plsc.load_gather(stg,
                                                 [gsp, zero + 1, lane, hsp[h]])
                mean16 = s16 * (1.0 / HIDDEN)
                var16 = t16 * (1.0 / HIDDEN) - mean16 * mean16
                rs16 = _rsqrt(var16 + LN_EPS)
                am16 = rs16 * msk16
                mam16 = mean16 * am16
                tv0 = zero + g * LANES
                for k in range(LANES):
                    amk = am16[k]
                    mamk = mam16[k]
                    mk = msk16[k]
                    tvec = tv0 + k
                    x = [rows[gbase + k, pl.ds(i * LANES, LANES)] + pq[i]
                         for i in range(HQ)]
                    for i in range(HQ):
                        u = x[i] * amk - mamk
                        y = u * gvec[i] + mk * bvec[i]
                        plsc.store_scatter(
                            outv, [livec, rowq[i], zero, lane_lo, tvec], y)

    def issue_wb(c, outv, sem):
        pltpu.async_copy(outv, out_hbm.at[pl.ds(c * CL, CL), :,
                                          pl.ds(wid, 1)], sem)

    def wait_wb(outv, sem):
        pltpu.make_async_copy(outv, out_hbm.at[pl.ds(0, CL), :,
                                               pl.ds(wid, 1)], sem).wait()

    issue_gather(0, idxa, rowsa, sga)
    issue_gather(1, idxb, rowsb, sgb)

    def body(k, carry):
        c = 2 * k
        wait_gather(idxa, rowsa, sga)

        @pl.when(k > 0)
        def _():
            wait_wb(outa, swa)

        compute(c, idxa, rowsa, outa)
        issue_wb(c, outa, swa)

        @pl.when(k < NCHUNK // 2 - 1)
        def _():
            issue_gather(c + 2, idxa, rowsa, sga)

        wait_gather(idxb, rowsb, sgb)

        @pl.when(k > 0)
        def _():
            wait_wb(outb, swb)

        compute(c + 1, idxb, rowsb, outb)
        issue_wb(c + 1, outb, swb)

        @pl.when(k < NCHUNK // 2 - 1)
        def _():
            issue_gather(c + 3, idxb, rowsb, sgb)

        return carry

    lax.fori_loop(0, NCHUNK // 2, body, 0)
    wait_wb(outa, swa)
    wait_wb(outb, swb)


_emb = functools.partial(
    pl.kernel,
    out_type=jax.ShapeDtypeStruct((SEQ_LEN, 8, NW, 8, BBLK), jnp.float32),
    mesh=plsc.VectorSubcoreMesh(core_axis_name="c", subcore_axis_name="s",
                                num_cores=NC, num_subcores=NS),
    compiler_params=pltpu.CompilerParams(needs_layout_passes=False,
                                         use_tc_tiling_on_sc=False),
    scratch_types=[
        pltpu.VMEM((CTOK + LANES,), jnp.int32),
        pltpu.VMEM((CTOK + LANES,), jnp.int32),
        pltpu.VMEM((CTOK, HIDDEN), jnp.float32),
        pltpu.VMEM((CTOK, HIDDEN), jnp.float32),
        pltpu.VMEM((CL, 8, 1, 8, BBLK), jnp.float32),
        pltpu.VMEM((CL, 8, 1, 8, BBLK), jnp.float32),
        pltpu.VMEM((SEQ_LEN * HIDDEN,), jnp.float32),
        pltpu.VMEM((NGRP, 2, LANES, LANES), jnp.float32),
        pltpu.VMEM((HIDDEN,), jnp.float32),
        pltpu.VMEM((HIDDEN,), jnp.float32),
        pltpu.SemaphoreType.DMA,
        pltpu.SemaphoreType.DMA,
        pltpu.SemaphoreType.DMA,
        pltpu.SemaphoreType.DMA,
    ],
)(_emb_body)


@jax.jit
def kernel(tokens, word_table, pos_table, gamma, beta):
    tok_t = tokens.T.astype(jnp.int32)
    out = _emb(tok_t, word_table, pos_table.reshape(-1), gamma, beta)
    z = out.transpose(2, 4, 0, 1, 3)
    return z.reshape(BATCH, SEQ_LEN, HIDDEN)


# speedup vs baseline: 1.2154x; 1.2154x over previous
import functools

import jax
import jax.numpy as jnp
from jax import lax
from jax.experimental import pallas as pl
from jax.experimental.pallas import tpu as pltpu
from jax.experimental.pallas import tpu_sc as plsc

VOCAB = 1000000
HIDDEN = 64
SEQ_LEN = 200
BATCH = 4096
N = BATCH * SEQ_LEN
NC, NS, LANES = 2, 16, 16
NW = NC * NS
BBLK = BATCH // NW
HQ = HIDDEN // LANES
CL = 2
CTOK = CL * BBLK
NCHUNK = SEQ_LEN // CL
NGRP = BBLK // LANES
LN_EPS = 1e-8


def _rsqrt(x):
    i = lax.bitcast_convert_type(x, jnp.int32)
    i = jnp.int32(0x5F3759DF) - lax.shift_right_logical(i, 1)
    y = lax.bitcast_convert_type(i, jnp.float32)
    for _ in range(3):
        y = y * (1.5 - 0.5 * x * y * y)
    return y


def _emb_body(tok_hbm, word_hbm, pos_hbm, gam_hbm, bet_hbm, out_hbm,
              tokv_all, rowsa, rowsb, outa, outb,
              pos_v, gv, bv, sga, sgb, swa, swb):
    wid = lax.axis_index("s") * NC + lax.axis_index("c")
    b0 = wid * BBLK

    pltpu.sync_copy(pos_hbm, pos_v)
    pltpu.sync_copy(tok_hbm.at[:, pl.ds(b0, BBLK)],
                    tokv_all.at[:, pl.ds(0, BBLK)])
    pltpu.sync_copy(gam_hbm, gv)
    pltpu.sync_copy(bet_hbm, bv)

    gvec = [gv[pl.ds(i * LANES, LANES)] for i in range(HQ)]
    bvec = [bv[pl.ds(i * LANES, LANES)] for i in range(HQ)]
    lane = lax.iota(jnp.int32, LANES)
    lane_hi = lax.shift_right_logical(lane, 3)
    lane_lo = lane & 7
    rowq = [lane_hi + 2 * q for q in range(HQ)]
    zero = lane & 0
    hsp = [zero + h for h in range(LANES)]

    def issue_gather(c, rows, sem):
        l0 = c * CL
        for li in range(CL):
            pltpu.async_copy(
                word_hbm.at[tokv_all.at[l0 + li, pl.ds(0, BBLK)]],
                rows.at[pl.ds(li * BBLK, BBLK)], sem)

    def wait_gather(rows, sem):
        for li in range(CL):
            pltpu.make_async_copy(
                word_hbm.at[tokv_all.at[0, pl.ds(0, BBLK)]],
                rows.at[pl.ds(li * BBLK, BBLK)], sem).wait()

    def compute(c, rows, outv):
        l0 = c * CL
        for li in range(CL):
            pq = [pos_v[pl.ds((l0 + li) * HIDDEN + i * LANES, LANES)]
                  for i in range(HQ)]
            livec = zero + li

            @plsc.parallel_loop(0, BBLK, 1, unroll=4)
            def token_body(t):
                g = li * BBLK + t
                tok = tokv_all[l0 + li, pl.ds(t, LANES)][0]
                x = [rows[g, pl.ds(i * LANES, LANES)] + pq[i]
                     for i in range(HQ)]
                s = (x[0] + x[1]) + (x[2] + x[3])
                ss = (x[0] * x[0] + x[1] * x[1]) + (x[2] * x[2] + x[3] * x[3])
                mean = jnp.sum(s) * (1.0 / HIDDEN)
                var = jnp.sum(ss) * (1.0 / HIDDEN) - mean * mean
                rs = _rsqrt(var + LN_EPS)
                msk = jnp.where(tok != 0, jnp.float32(1.0), jnp.float32(0.0))
                rsm = rs * msk
                tvec = zero + t
                for i in range(HQ):
                    y = (x[i] - mean) * rsm * gvec[i] + msk * bvec[i]
                    plsc.store_scatter(
                        outv, [livec, rowq[i], zero, lane_lo, tvec], y)

    def issue_wb(c, outv, sem):
        pltpu.async_copy(outv, out_hbm.at[pl.ds(c * CL, CL), :,
                                          pl.ds(wid, 1)], sem)

    def wait_wb(outv, sem):
        pltpu.make_async_copy(outv, out_hbm.at[pl.ds(0, CL), :,
                                               pl.ds(wid, 1)], sem).wait()

    issue_gather(0, rowsa, sga)
    issue_gather(1, rowsb, sgb)

    def body(k, carry):
        c = 2 * k
        wait_gather(rowsa, sga)

        @pl.when(k > 0)
        def _():
            wait_wb(outa, swa)

        compute(c, rowsa, outa)
        issue_wb(c, outa, swa)

        @pl.when(k < NCHUNK // 2 - 1)
        def _():
            issue_gather(c + 2, rowsa, sga)

        wait_gather(rowsb, sgb)

        @pl.when(k > 0)
        def _():
            wait_wb(outb, swb)

        compute(c + 1, rowsb, outb)
        issue_wb(c + 1, outb, swb)

        @pl.when(k < NCHUNK // 2 - 1)
        def _():
            issue_gather(c + 3, rowsb, sgb)

        return carry

    lax.fori_loop(0, NCHUNK // 2, body, 0)
    wait_wb(outa, swa)
    wait_wb(outb, swb)


_emb = functools.partial(
    pl.kernel,
    out_type=jax.ShapeDtypeStruct((SEQ_LEN, 8, NW, 8, BBLK), jnp.float32),
    mesh=plsc.VectorSubcoreMesh(core_axis_name="c", subcore_axis_name="s",
                                num_cores=NC, num_subcores=NS),
    compiler_params=pltpu.CompilerParams(needs_layout_passes=False,
                                         use_tc_tiling_on_sc=False),
    scratch_types=[
        pltpu.VMEM((SEQ_LEN, 144), jnp.int32),
        pltpu.VMEM((CTOK, HIDDEN), jnp.float32),
        pltpu.VMEM((CTOK, HIDDEN), jnp.float32),
        pltpu.VMEM((CL, 8, 1, 8, BBLK), jnp.float32),
        pltpu.VMEM((CL, 8, 1, 8, BBLK), jnp.float32),
        pltpu.VMEM((SEQ_LEN * HIDDEN,), jnp.float32),
        pltpu.VMEM((HIDDEN,), jnp.float32),
        pltpu.VMEM((HIDDEN,), jnp.float32),
        pltpu.SemaphoreType.DMA,
        pltpu.SemaphoreType.DMA,
        pltpu.SemaphoreType.DMA,
        pltpu.SemaphoreType.DMA,
    ],
)(_emb_body)


@jax.jit
def kernel(tokens, word_table, pos_table, gamma, beta):
    tok_t = tokens.T.astype(jnp.int32)
    out = _emb(tok_t, word_table, pos_table.reshape(-1), gamma, beta)
    z = out.transpose(2, 4, 0, 1, 3)
    return z.reshape(BATCH, SEQ_LEN, HIDDEN)


# speedup vs baseline: 1.2252x; 1.0080x over previous
import functools

import jax
import jax.numpy as jnp
from jax import lax
from jax.experimental import pallas as pl
from jax.experimental.pallas import tpu as pltpu
from jax.experimental.pallas import tpu_sc as plsc

VOCAB = 1000000
HIDDEN = 64
SEQ_LEN = 200
BATCH = 4096
N = BATCH * SEQ_LEN
NC, NS, LANES = 2, 16, 16
NW = NC * NS
BBLK = BATCH // NW
HQ = HIDDEN // LANES
CL = 2
CTOK = CL * BBLK
NCHUNK = SEQ_LEN // CL
NGRP = BBLK // LANES
LN_EPS = 1e-8


def _rsqrt(x):
    i = lax.bitcast_convert_type(x, jnp.int32)
    i = jnp.int32(0x5F3759DF) - lax.shift_right_logical(i, 1)
    y = lax.bitcast_convert_type(i, jnp.float32)
    for _ in range(3):
        y = y * (1.5 - 0.5 * x * y * y)
    return y


def _emb_body(tok_hbm, word_hbm, pos_hbm, gam_hbm, bet_hbm, out_hbm,
              tokv_all, rowsa, rowsb, outa, outb,
              pos_v, gv, bv, sga, sgb, swa, swb):
    wid = lax.axis_index("s") * NC + lax.axis_index("c")
    b0 = wid * BBLK

    pltpu.sync_copy(pos_hbm, pos_v)
    pltpu.sync_copy(tok_hbm.at[:, pl.ds(b0, BBLK)],
                    tokv_all.at[:, pl.ds(0, BBLK)])
    pltpu.sync_copy(gam_hbm, gv)
    pltpu.sync_copy(bet_hbm, bv)

    gvec = [gv[pl.ds(i * LANES, LANES)] for i in range(HQ)]
    bvec = [bv[pl.ds(i * LANES, LANES)] for i in range(HQ)]
    lane = lax.iota(jnp.int32, LANES)
    lane_hi = lax.shift_right_logical(lane, 3)
    lane_lo = lane & 7
    rowq = [lane_hi + 2 * q for q in range(HQ)]
    zero = lane & 0
    hsp = [zero + h for h in range(LANES)]

    def issue_gather(c, rows, sem):
        l0 = c * CL
        for li in range(CL):
            pltpu.async_copy(
                word_hbm.at[tokv_all.at[l0 + li, pl.ds(0, BBLK)]],
                rows.at[pl.ds(li * BBLK, BBLK)], sem)

    def wait_gather(rows, sem):
        for li in range(CL):
            pltpu.make_async_copy(
                word_hbm.at[tokv_all.at[0, pl.ds(0, BBLK)]],
                rows.at[pl.ds(li * BBLK, BBLK)], sem).wait()

    def compute(c, rows, outv):
        l0 = c * CL

        @plsc.parallel_loop(0, CTOK, 1, unroll=4)
        def token_body(g):
            li = lax.shift_right_logical(g, 7)
            t = g & (BBLK - 1)
            l = l0 + li
            tok = tokv_all[l, pl.ds(t, LANES)][0]
            x = [rows[g, pl.ds(i * LANES, LANES)]
                 + pos_v[pl.ds(l * HIDDEN + i * LANES, LANES)]
                 for i in range(HQ)]
            s = (x[0] + x[1]) + (x[2] + x[3])
            ss = (x[0] * x[0] + x[1] * x[1]) + (x[2] * x[2] + x[3] * x[3])
            mean = jnp.sum(s) * (1.0 / HIDDEN)
            var = jnp.sum(ss) * (1.0 / HIDDEN) - mean * mean
            rs = _rsqrt(var + LN_EPS)
            msk = jnp.where(tok != 0, jnp.float32(1.0), jnp.float32(0.0))
            rsm = rs * msk
            tvec = zero + t
            livec = zero + li
            for i in range(HQ):
                y = (x[i] - mean) * rsm * gvec[i] + msk * bvec[i]
                plsc.store_scatter(
                    outv, [livec, rowq[i], zero, lane_lo, tvec], y)

    def issue_wb(c, outv, sem):
        pltpu.async_copy(outv, out_hbm.at[pl.ds(c * CL, CL), :,
                                          pl.ds(wid, 1)], sem)

    def wait_wb(outv, sem):
        pltpu.make_async_copy(outv, out_hbm.at[pl.ds(0, CL), :,
                                               pl.ds(wid, 1)], sem).wait()

    issue_gather(0, rowsa, sga)
    issue_gather(1, rowsb, sgb)

    def body(k, carry):
        c = 2 * k
        wait_gather(rowsa, sga)

        @pl.when(k > 0)
        def _():
            wait_wb(outa, swa)

        compute(c, rowsa, outa)
        issue_wb(c, outa, swa)

        @pl.when(k < NCHUNK // 2 - 1)
        def _():
            issue_gather(c + 2, rowsa, sga)

        wait_gather(rowsb, sgb)

        @pl.when(k > 0)
        def _():
            wait_wb(outb, swb)

        compute(c + 1, rowsb, outb)
        issue_wb(c + 1, outb, swb)

        @pl.when(k < NCHUNK // 2 - 1)
        def _():
            issue_gather(c + 3, rowsb, sgb)

        return carry

    lax.fori_loop(0, NCHUNK // 2, body, 0)
    wait_wb(outa, swa)
    wait_wb(outb, swb)


_emb = functools.partial(
    pl.kernel,
    out_type=jax.ShapeDtypeStruct((SEQ_LEN, 8, NW, 8, BBLK), jnp.float32),
    mesh=plsc.VectorSubcoreMesh(core_axis_name="c", subcore_axis_name="s",
                                num_cores=NC, num_subcores=NS),
    compiler_params=pltpu.CompilerParams(needs_layout_passes=False,
                                         use_tc_tiling_on_sc=False),
    scratch_types=[
        pltpu.VMEM((SEQ_LEN, 144), jnp.int32),
        pltpu.VMEM((CTOK, HIDDEN), jnp.float32),
        pltpu.VMEM((CTOK, HIDDEN), jnp.float32),
        pltpu.VMEM((CL, 8, 1, 8, BBLK), jnp.float32),
        pltpu.VMEM((CL, 8, 1, 8, BBLK), jnp.float32),
        pltpu.VMEM((SEQ_LEN * HIDDEN,), jnp.float32),
        pltpu.VMEM((HIDDEN,), jnp.float32),
        pltpu.VMEM((HIDDEN,), jnp.float32),
        pltpu.SemaphoreType.DMA,
        pltpu.SemaphoreType.DMA,
        pltpu.SemaphoreType.DMA,
        pltpu.SemaphoreType.DMA,
    ],
)(_emb_body)


@jax.jit
def kernel(tokens, word_table, pos_table, gamma, beta):
    tok_t = tokens.T.astype(jnp.int32)
    out = _emb(tok_t, word_table, pos_table.reshape(-1), gamma, beta)
    z = out.transpose(2, 4, 0, 1, 3)
    return z.reshape(BATCH, SEQ_LEN, HIDDEN)
